# baseline (speedup 1.0000x reference)
import functools

import jax
import jax.numpy as jnp
from jax.experimental import pallas as pl
from jax.experimental.pallas import tpu as pltpu

T = 2048
D = 1024
I = 512
E = 16
K = 8
N_GROUP = 8
TOPK_GROUP = 4
RSF = 2.5
NE = E + 1
WCOLS = 32


def _routing_body(x_ref, gw_ref, gb_ref, wt_ref):
    xb = x_ref[...].astype(jnp.bfloat16)
    gwb = gw_ref[...].astype(jnp.bfloat16)
    logits = jax.lax.dot_general(
        xb, gwb, (((1,), (1,)), ((), ())),
        preferred_element_type=jnp.float32)
    scores = jax.nn.sigmoid(logits)
    sfc = scores + gb_ref[...]

    ecol = jax.lax.broadcasted_iota(jnp.int32, (T, E), 1)
    gs = jnp.concatenate(
        [jnp.sum(jnp.where(ecol // 2 == g, sfc, 0.0), axis=1, keepdims=True)
         for g in range(N_GROUP)], axis=1)

    col = jax.lax.broadcasted_iota(jnp.int32, (T, N_GROUP), 1)
    rank = jnp.zeros((T, N_GROUP), jnp.int32)
    for gp in range(N_GROUP):
        other = gs[:, gp:gp + 1]
        gt = (other > gs).astype(jnp.int32)
        tie = jnp.logical_and(other == gs, gp < col).astype(jnp.int32)
        rank = rank + gt + tie
    gmask = (rank < TOPK_GROUP).astype(jnp.float32)

    e_iota = jax.lax.broadcasted_iota(jnp.int32, (N_GROUP, E), 0)
    g_iota = jax.lax.broadcasted_iota(jnp.int32, (N_GROUP, E), 1)
    pmt = (g_iota // 2 == e_iota).astype(jnp.float32)
    emask = jax.lax.dot_general(
        gmask, pmt, (((1,), (0,)), ((), ())),
        preferred_element_type=jnp.float32)

    w = scores * emask
    denom = jnp.sum(w, axis=1, keepdims=True) + 1e-20
    w = w * (RSF / denom)

    tail_iota = jax.lax.broadcasted_iota(jnp.int32, (T, WCOLS - E), 1)
    tail = (tail_iota == 0).astype(jnp.float32)
    wt_ref[...] = jnp.concatenate([w, tail], axis=1)


def _swiglu(x, w1, w3, w2, scale):
    u = jax.lax.dot_general(
        x, w1.astype(jnp.bfloat16), (((1,), (1,)), ((), ())),
        preferred_element_type=jnp.float32)
    v = jax.lax.dot_general(
        x, w3.astype(jnp.bfloat16), (((1,), (1,)), ((), ())),
        preferred_element_type=jnp.float32)
    h = (u * jax.nn.sigmoid(u)) * v * scale
    return jax.lax.dot_general(
        h.astype(jnp.bfloat16), w2.astype(jnp.bfloat16),
        (((1,), (1,)), ((), ())),
        preferred_element_type=jnp.float32)


def _expert_body(wt_ref, x_ref, w1_ref, w3_ref, w2_ref, sw1_ref, sw3_ref,
                 sw2_ref, out_ref):
    e = pl.program_id(0)

    @pl.when(e < E)
    def _routed():
        sel = (jax.lax.broadcasted_iota(jnp.int32, (1, WCOLS), 1) == e)
        we = jnp.sum(wt_ref[...] * sel.astype(jnp.float32), axis=1,
                     keepdims=True)
        y = _swiglu(x_ref[...], w1_ref[0], w3_ref[0], w2_ref[0], we)

        @pl.when(e == 0)
        def _init():
            out_ref[...] = y

        @pl.when(e != 0)
        def _acc():
            out_ref[...] += y

    @pl.when(e == E)
    def _shared():
        one = jnp.ones((T, 1), jnp.float32)
        out_ref[...] += _swiglu(x_ref[...], sw1_ref[...], sw3_ref[...],
                                sw2_ref[...], one)


@jax.jit
def kernel(x, gate_w, gate_bias, w1, w3, w2, sw1, sw3, sw2):
    wt = pl.pallas_call(
        _routing_body,
        out_shape=jax.ShapeDtypeStruct((T, WCOLS), jnp.float32),
    )(x, gate_w, gate_bias.reshape(1, E))

    xb = x.astype(jnp.bfloat16)

    out = pl.pallas_call(
        _expert_body,
        grid=(E + 1,),
        in_specs=[
            pl.BlockSpec((T, WCOLS), lambda e: (0, 0)),
            pl.BlockSpec((T, D), lambda e: (0, 0)),
            pl.BlockSpec((1, I, D), lambda e: (jnp.minimum(e, E - 1), 0, 0)),
            pl.BlockSpec((1, I, D), lambda e: (jnp.minimum(e, E - 1), 0, 0)),
            pl.BlockSpec((1, D, I), lambda e: (jnp.minimum(e, E - 1), 0, 0)),
            pl.BlockSpec((I, D), lambda e: (0, 0)),
            pl.BlockSpec((I, D), lambda e: (0, 0)),
            pl.BlockSpec((D, I), lambda e: (0, 0)),
        ],
        out_specs=pl.BlockSpec((T, D), lambda e: (0, 0)),
        out_shape=jax.ShapeDtypeStruct((T, D), jnp.float32),
    )(wt, xb, w1, w3, w2, sw1, sw3, sw2)
    return out


# speedup vs baseline: 1.0714x; 1.0714x over previous
import functools

import jax
import jax.numpy as jnp
from jax.experimental import pallas as pl
from jax.experimental.pallas import tpu as pltpu

T = 2048
D = 1024
I = 512
E = 16
K = 8
N_GROUP = 8
TOPK_GROUP = 4
RSF = 2.5
NE = E + 1
WCOLS = 32


def _routing_body(x_ref, gw_ref, gb_ref, wt_ref):
    xb = x_ref[...].astype(jnp.bfloat16)
    gwb = gw_ref[...].astype(jnp.bfloat16)
    logits = jax.lax.dot_general(
        xb, gwb, (((1,), (1,)), ((), ())),
        preferred_element_type=jnp.float32)
    scores = jax.nn.sigmoid(logits)
    sfc = scores + gb_ref[...]

    ecol = jax.lax.broadcasted_iota(jnp.int32, (T, E), 1)
    gs = jnp.concatenate(
        [jnp.sum(jnp.where(ecol // 2 == g, sfc, 0.0), axis=1, keepdims=True)
         for g in range(N_GROUP)], axis=1)

    col = jax.lax.broadcasted_iota(jnp.int32, (T, N_GROUP), 1)
    rank = jnp.zeros((T, N_GROUP), jnp.int32)
    for gp in range(N_GROUP):
        other = gs[:, gp:gp + 1]
        gt = (other > gs).astype(jnp.int32)
        tie = jnp.logical_and(other == gs, gp < col).astype(jnp.int32)
        rank = rank + gt + tie
    gmask = (rank < TOPK_GROUP).astype(jnp.float32)

    e_iota = jax.lax.broadcasted_iota(jnp.int32, (N_GROUP, E), 0)
    g_iota = jax.lax.broadcasted_iota(jnp.int32, (N_GROUP, E), 1)
    pmt = (g_iota // 2 == e_iota).astype(jnp.float32)
    emask = jax.lax.dot_general(
        gmask, pmt, (((1,), (0,)), ((), ())),
        preferred_element_type=jnp.float32)

    w = scores * emask
    denom = jnp.sum(w, axis=1, keepdims=True) + 1e-20
    w = w * (RSF / denom)

    tail_iota = jax.lax.broadcasted_iota(jnp.int32, (T, WCOLS - E), 1)
    tail = (tail_iota == 0).astype(jnp.float32)
    wt_ref[...] = jnp.concatenate([w, tail], axis=1)


def _swiglu(x, w1, w3, w2, scale):
    u = jax.lax.dot_general(
        x, w1.astype(jnp.bfloat16), (((1,), (1,)), ((), ())),
        preferred_element_type=jnp.float32)
    v = jax.lax.dot_general(
        x, w3.astype(jnp.bfloat16), (((1,), (1,)), ((), ())),
        preferred_element_type=jnp.float32)
    h = (u * jax.nn.sigmoid(u)) * v * scale
    return jax.lax.dot_general(
        h.astype(jnp.bfloat16), w2.astype(jnp.bfloat16),
        (((1,), (1,)), ((), ())),
        preferred_element_type=jnp.float32)


def _expert_body(wt_ref, x_ref, w1_ref, w3_ref, w2_ref, sw1_ref, sw3_ref,
                 sw2_ref, out_ref):
    e = pl.program_id(0)

    @pl.when(e == 0)
    def _init():
        out_ref[...] = jnp.zeros((T, D), jnp.float32)

    @pl.when(e < E)
    def _routed():
        sel = (jax.lax.broadcasted_iota(jnp.int32, (1, WCOLS), 1) == e)
        we = jnp.sum(wt_ref[...] * sel.astype(jnp.float32), axis=1,
                     keepdims=True)
        out_ref[...] += _swiglu(x_ref[...], w1_ref[0], w3_ref[0], w2_ref[0],
                                we)

    @pl.when(e == E)
    def _shared():
        one = jnp.ones((T, 1), jnp.float32)
        out_ref[...] += _swiglu(x_ref[...], sw1_ref[...], sw3_ref[...],
                                sw2_ref[...], one)


@jax.jit
def kernel(x, gate_w, gate_bias, w1, w3, w2, sw1, sw3, sw2):
    wt = pl.pallas_call(
        _routing_body,
        out_shape=jax.ShapeDtypeStruct((T, WCOLS), jnp.float32),
    )(x, gate_w, gate_bias.reshape(1, E))

    xb = x.astype(jnp.bfloat16)

    out = pl.pallas_call(
        _expert_body,
        grid=(E + 1,),
        in_specs=[
            pl.BlockSpec((T, WCOLS), lambda e: (0, 0)),
            pl.BlockSpec((T, D), lambda e: (0, 0)),
            pl.BlockSpec((1, I, D), lambda e: (jnp.minimum(e, E - 1), 0, 0)),
            pl.BlockSpec((1, I, D), lambda e: (jnp.minimum(e, E - 1), 0, 0)),
            pl.BlockSpec((1, D, I), lambda e: (jnp.minimum(e, E - 1), 0, 0)),
            pl.BlockSpec((I, D), lambda e: (0, 0)),
            pl.BlockSpec((I, D), lambda e: (0, 0)),
            pl.BlockSpec((D, I), lambda e: (0, 0)),
        ],
        out_specs=pl.BlockSpec((T, D), lambda e: (0, 0)),
        out_shape=jax.ShapeDtypeStruct((T, D), jnp.float32),
    )(wt, xb, w1, w3, w2, sw1, sw3, sw2)
    return out
